# baseline (speedup 1.0000x reference)
import functools

import jax
import jax.numpy as jnp
from jax import lax
from jax.experimental import pallas as pl

B, N, C, K, H, W = 16, 18525, 81, 32, 136, 136
TOPN, MAX_OBJ = 200, 100
MIN_SCORE, NMS_THR = 0.05, 0.5


def _pairwise_iou(b):
    x1 = jnp.maximum(b[:, :, None, 0], b[:, None, :, 0])
    y1 = jnp.maximum(b[:, :, None, 1], b[:, None, :, 1])
    x2 = jnp.minimum(b[:, :, None, 2], b[:, None, :, 2])
    y2 = jnp.minimum(b[:, :, None, 3], b[:, None, :, 3])
    inter = jnp.clip(x2 - x1, 0.0) * jnp.clip(y2 - y1, 0.0)
    area = (b[..., 2] - b[..., 0]) * (b[..., 3] - b[..., 1])
    union = area[:, :, None] + area[:, None, :] - inter
    return inter / jnp.maximum(union, 1e-9)


def _decode_one(cls_logits, box_p, coef_p, proto, anchors):
    p = jax.nn.softmax(cls_logits, axis=-1)
    cls = p[:, 1:]
    valid = jnp.max(cls, axis=1) > MIN_SCORE
    cls = cls * valid[:, None].astype(cls.dtype)

    xy = anchors[:, :2] + box_p[:, :2] * 0.1 * anchors[:, 2:4]
    wh = anchors[:, 2:4] * jnp.exp(box_p[:, 2:4] * 0.2)
    x1y1 = xy - wh / 2.0
    boxes = jnp.clip(jnp.concatenate([x1y1, x1y1 + wh], axis=1), 0.0, 1.0)

    v200 = -jnp.sort(-cls, axis=0)[TOPN - 1]
    gt = cls > v200[None, :]
    eq = cls == v200[None, :]
    m = jnp.sum(gt, axis=0)
    r = TOPN - m
    eq_rank = jnp.cumsum(eq.astype(jnp.int32), axis=0)
    sel = gt | (eq & (eq_rank <= r[None, :]))

    idx_col = jnp.where(sel, jnp.arange(N)[:, None], N)
    sel_idx = jnp.sort(idx_col, axis=0)[:TOPN].T
    v = jnp.take_along_axis(cls.T, sel_idx, axis=1)
    b = boxes[sel_idx]
    co = coef_p[sel_idx]

    iou = _pairwise_iou(b)
    prec = (v[:, :, None] > v[:, None, :]) | (
        (v[:, :, None] == v[:, None, :]) & (sel_idx[:, :, None] < sel_idx[:, None, :]))
    suppressed = jnp.any(prec & (iou > NMS_THR), axis=1)
    keep = ~suppressed

    scores_f = (v * keep.astype(v.dtype) * (v > MIN_SCORE).astype(v.dtype)).reshape(-1)
    fs, fi = lax.top_k(scores_f, MAX_OBJ)
    fb = b.reshape(-1, 4)[fi]
    fc = co.reshape(-1, K)[fi]
    fl = fi // TOPN

    logit = proto.reshape(-1, K) @ fc.T
    logit = logit.reshape(H, W, MAX_OBJ)
    cols = jnp.arange(W)[None, :, None].astype(jnp.float32)
    rows = jnp.arange(H)[:, None, None].astype(jnp.float32)
    inside = (cols >= fb[:, 0] * W) & (cols < fb[:, 2] * W) & \
             (rows >= fb[:, 1] * H) & (rows < fb[:, 3] * H)
    masks = ((logit > 0.0) & inside).astype(jnp.float32)
    masks = jnp.transpose(masks, (2, 0, 1))
    return masks, fl.astype(jnp.int32), fs


def _noop_body(x_ref, o_ref):
    o_ref[...] = x_ref[...]


def kernel(class_preds, box_preds, coef_preds, proto_outs, anchors):
    anchors = pl.pallas_call(
        _noop_body,
        out_shape=jax.ShapeDtypeStruct(anchors.shape, anchors.dtype),
    )(anchors)
    return jax.vmap(_decode_one, in_axes=(0, 0, 0, 0, None))(
        class_preds, box_preds, coef_preds, proto_outs, anchors)


# speedup vs baseline: 2.1493x; 2.1493x over previous
import functools

import jax
import jax.numpy as jnp
from jax import lax
from jax.experimental import pallas as pl
from jax.experimental.pallas import tpu as pltpu

B, N, C, K, H, W = 16, 18525, 81, 32, 136, 136
TOPN, MAX_OBJ = 200, 100
MIN_SCORE, NMS_THR = 0.05, 0.5


CH = 1544
NBLK = 12
NP = CH * NBLK


def _stage1a_body(cls_ref, box_ref, anc_ref, p_out, boxes_out):
    j = pl.program_id(1)
    row0 = j * CH
    x = cls_ref[0]
    xm = jnp.max(x, axis=1, keepdims=True)
    e = jnp.exp(x - xm)
    s = jnp.sum(e, axis=1, keepdims=True)
    p = e / s
    li = jax.lax.broadcasted_iota(jnp.int32, (CH, C), 1)
    pm = jnp.where(li >= 1, p, 0.0)
    valid = jnp.max(pm, axis=1, keepdims=True) > MIN_SCORE
    ri = row0 + jax.lax.broadcasted_iota(jnp.int32, (CH, C), 0)
    p_out[0] = jnp.where(ri < N, pm * valid.astype(pm.dtype), 0.0)

    bp = box_ref[0]
    anc = anc_ref[...]
    xy = anc[:, :2] + bp[:, :2] * 0.1 * anc[:, 2:4]
    wh = anc[:, 2:4] * jnp.exp(bp[:, 2:4] * 0.2)
    x1y1 = xy - wh / 2.0
    bx = jnp.clip(jnp.concatenate([x1y1, x1y1 + wh], axis=1), 0.0, 1.0)
    ri4 = row0 + jax.lax.broadcasted_iota(jnp.int32, (CH, 4), 0)
    boxes_out[0] = jnp.where(ri4 < N, bx, 0.0)


def _stage1b_body(p_hbm, meta_out, scratch, sem, *, topn=TOPN):
    b = pl.program_id(0)
    cp = pltpu.make_async_copy(p_hbm.at[b], scratch, sem)
    cp.start()
    cp.wait()

    def count_gt(tf):
        def blk(k, acc):
            ch = scratch[pl.ds(k * CH, CH), :]
            return acc + jnp.sum((ch > tf).astype(jnp.int32), axis=0,
                                 keepdims=True)
        return jax.lax.fori_loop(0, NBLK, blk, jnp.zeros((1, C), jnp.int32))

    one_bits = jnp.int32(0x3F800000)
    lo0 = jnp.zeros((1, C), jnp.int32)
    hi0 = jnp.full((1, C), one_bits, jnp.int32)

    def bs_body(_, lohi):
        lo, hi = lohi
        mid = (lo + hi) >> 1
        midf = jax.lax.bitcast_convert_type(mid, jnp.float32)
        pred = count_gt(midf) >= topn
        return jnp.where(pred, mid, lo), jnp.where(pred, hi, mid)

    lo, hi = jax.lax.fori_loop(0, 31, bs_body, (lo0, hi0))
    cnt0 = count_gt(jnp.zeros((1, C), jnp.float32))
    v200b = jnp.where(cnt0 >= topn, hi, 0)
    v200f = jax.lax.bitcast_convert_type(v200b, jnp.float32)
    m = count_gt(v200f)
    r = topn - m

    def cnt_le(I):
        def blk(k, acc):
            ch = scratch[pl.ds(k * CH, CH), :]
            ai = k * CH + jax.lax.broadcasted_iota(jnp.int32, (CH, C), 0)
            hit = (ch == v200f) & (ai <= I)
            return acc + jnp.sum(hit.astype(jnp.int32), axis=0, keepdims=True)
        return jax.lax.fori_loop(0, NBLK, blk, jnp.zeros((1, C), jnp.int32))

    def bsI_body(_, lohi):
        lo, hi = lohi
        mid = (lo + hi) >> 1
        pred = cnt_le(mid) >= r
        return jnp.where(pred, lo, mid), jnp.where(pred, mid, hi)

    loI0 = jnp.full((1, C), -1, jnp.int32)
    hiI0 = jnp.full((1, C), N - 1, jnp.int32)
    loI, hiI = jax.lax.fori_loop(0, 15, bsI_body, (loI0, hiI0))
    Ilim = jnp.where(r > 0, hiI, -1)

    zero = jnp.zeros((1, C), jnp.int32)
    meta_out[0] = jnp.concatenate(
        [v200b, Ilim, r, m, zero, zero, zero, zero], axis=0)


def _stage1_body(cls_ref, box_ref, anc_ref, p_out, meta_out, boxes_out,
                 *, n=N, topn=TOPN):
    x = cls_ref[0]
    xm = jnp.max(x, axis=1, keepdims=True)
    e = jnp.exp(x - xm)
    s = jnp.sum(e, axis=1, keepdims=True)
    p = e / s
    li = jax.lax.broadcasted_iota(jnp.int32, (n, C), 1)
    pm = jnp.where(li >= 1, p, 0.0)
    valid = jnp.max(pm, axis=1, keepdims=True) > MIN_SCORE
    pmm = pm * valid.astype(pm.dtype)
    p_out[0] = pmm

    bp = box_ref[0]
    anc = anc_ref[...]
    xy = anc[:, :2] + bp[:, :2] * 0.1 * anc[:, 2:4]
    wh = anc[:, 2:4] * jnp.exp(bp[:, 2:4] * 0.2)
    x1y1 = xy - wh / 2.0
    boxes_out[0] = jnp.clip(jnp.concatenate([x1y1, x1y1 + wh], axis=1), 0.0, 1.0)

    def count_gt(tf):
        return jnp.sum((pmm > tf).astype(jnp.int32), axis=0, keepdims=True)

    one_bits = jnp.int32(0x3F800000)
    lo0 = jnp.zeros((1, C), jnp.int32)
    hi0 = jnp.full((1, C), one_bits, jnp.int32)

    def bs_body(_, lohi):
        lo, hi = lohi
        mid = (lo + hi) >> 1
        midf = jax.lax.bitcast_convert_type(mid, jnp.float32)
        pred = count_gt(midf) >= topn
        return jnp.where(pred, mid, lo), jnp.where(pred, hi, mid)

    lo, hi = jax.lax.fori_loop(0, 31, bs_body, (lo0, hi0))
    cnt0 = count_gt(jnp.zeros((1, C), jnp.float32))
    v200b = jnp.where(cnt0 >= topn, hi, 0)
    v200f = jax.lax.bitcast_convert_type(v200b, jnp.float32)
    m = count_gt(v200f)
    r = topn - m

    eq = pmm == v200f
    ai = jax.lax.broadcasted_iota(jnp.int32, (n, C), 0)

    def bsI_body(_, lohi):
        lo, hi = lohi
        mid = (lo + hi) >> 1
        cnt = jnp.sum((eq & (ai <= mid)).astype(jnp.int32), axis=0, keepdims=True)
        pred = cnt >= r
        return jnp.where(pred, lo, mid), jnp.where(pred, mid, hi)

    loI0 = jnp.full((1, C), -1, jnp.int32)
    hiI0 = jnp.full((1, C), n - 1, jnp.int32)
    loI, hiI = jax.lax.fori_loop(0, 15, bsI_body, (loI0, hiI0))
    Ilim = jnp.where(r > 0, hiI, -1)

    zero = jnp.zeros((1, C), jnp.int32)
    meta_out[0] = jnp.concatenate(
        [v200b, Ilim, r, m, zero, zero, zero, zero], axis=0)


PIX = H * W
PIXP = 18560
PCH = 3712
NPBLK = 5


def _stage5_body(proto_ref, fc_ref, fb_ref, out_ref):
    j = pl.program_id(1)
    fc = fc_ref[0]
    pt = proto_ref[0]
    logit = jnp.dot(fc, pt)
    pix = j * PCH + jax.lax.broadcasted_iota(jnp.int32, (MAX_OBJ, PCH), 1)
    px = (pix % W).astype(jnp.float32)
    py = (pix // W).astype(jnp.float32)
    fb = fb_ref[0]
    x1 = fb[:, 0:1] * W
    y1 = fb[:, 1:2] * H
    x2 = fb[:, 2:3] * W
    y2 = fb[:, 3:4] * H
    inside = (px >= x1) & (px < x2) & (py >= y1) & (py < y2)
    out_ref[0] = ((logit > 0.0) & inside).astype(jnp.float32)


def _pairwise_iou(b):
    x1 = jnp.maximum(b[:, :, None, 0], b[:, None, :, 0])
    y1 = jnp.maximum(b[:, :, None, 1], b[:, None, :, 1])
    x2 = jnp.minimum(b[:, :, None, 2], b[:, None, :, 2])
    y2 = jnp.minimum(b[:, :, None, 3], b[:, None, :, 3])
    inter = jnp.clip(x2 - x1, 0.0) * jnp.clip(y2 - y1, 0.0)
    area = (b[..., 2] - b[..., 0]) * (b[..., 3] - b[..., 1])
    union = area[:, :, None] + area[:, None, :] - inter
    return inter / jnp.maximum(union, 1e-9)


def _decode_one(pmm81, thr_bits, Ilim, boxes, coef_p):
    cls = pmm81[:, 1:]
    v200 = jax.lax.bitcast_convert_type(thr_bits[1:], jnp.float32)
    gt = cls > v200[None, :]
    eq = cls == v200[None, :]
    sel = gt | (eq & (jnp.arange(N)[:, None] <= Ilim[None, 1:]))

    idx_col = jnp.where(sel, jnp.arange(N)[:, None], N)
    sel_idx = jnp.sort(idx_col, axis=0)[:TOPN].T
    v = jnp.take_along_axis(cls.T, sel_idx, axis=1)
    b = boxes[sel_idx]
    co = coef_p[sel_idx]

    iou = _pairwise_iou(b)
    prec = (v[:, :, None] > v[:, None, :]) | (
        (v[:, :, None] == v[:, None, :]) & (sel_idx[:, :, None] < sel_idx[:, None, :]))
    suppressed = jnp.any(prec & (iou > NMS_THR), axis=1)
    keep = ~suppressed

    scores_f = (v * keep.astype(v.dtype) * (v > MIN_SCORE).astype(v.dtype)).reshape(-1)
    fs, fi = lax.top_k(scores_f, MAX_OBJ)
    fb = b.reshape(-1, 4)[fi]
    fc = co.reshape(-1, K)[fi]
    fl = fi // TOPN
    return fb, fc, fl.astype(jnp.int32), fs


def kernel(class_preds, box_preds, coef_preds, proto_outs, anchors):
    p_pad, boxes_pad = pl.pallas_call(
        _stage1a_body,
        grid=(B, NBLK),
        in_specs=[
            pl.BlockSpec((1, CH, C), lambda i, j: (i, j, 0)),
            pl.BlockSpec((1, CH, 4), lambda i, j: (i, j, 0)),
            pl.BlockSpec((CH, 4), lambda i, j: (j, 0)),
        ],
        out_specs=[
            pl.BlockSpec((1, CH, C), lambda i, j: (i, j, 0)),
            pl.BlockSpec((1, CH, 4), lambda i, j: (i, j, 0)),
        ],
        out_shape=[
            jax.ShapeDtypeStruct((B, NP, C), jnp.float32),
            jax.ShapeDtypeStruct((B, NP, 4), jnp.float32),
        ],
    )(class_preds, box_preds, anchors)

    meta = pl.pallas_call(
        _stage1b_body,
        grid=(B,),
        in_specs=[pl.BlockSpec(memory_space=pltpu.MemorySpace.HBM)],
        out_specs=pl.BlockSpec((1, 8, C), lambda i: (i, 0, 0)),
        out_shape=jax.ShapeDtypeStruct((B, 8, C), jnp.int32),
        scratch_shapes=[
            pltpu.VMEM((NP, C), jnp.float32),
            pltpu.SemaphoreType.DMA,
        ],
    )(p_pad)

    pmm = p_pad[:, :N, :]
    boxes = boxes_pad[:, :N, :]
    fb, fc, fl, fs = jax.vmap(_decode_one)(
        pmm, meta[:, 0, :], meta[:, 1, :], boxes, coef_preds)

    proto_t = proto_outs.reshape(B, PIX, K).transpose(0, 2, 1)
    proto_t = jnp.pad(proto_t, ((0, 0), (0, 0), (0, PIXP - PIX)))
    masks = pl.pallas_call(
        _stage5_body,
        grid=(B, NPBLK),
        in_specs=[
            pl.BlockSpec((1, K, PCH), lambda i, j: (i, 0, j)),
            pl.BlockSpec((1, MAX_OBJ, K), lambda i, j: (i, 0, 0)),
            pl.BlockSpec((1, MAX_OBJ, 4), lambda i, j: (i, 0, 0)),
        ],
        out_specs=pl.BlockSpec((1, MAX_OBJ, PCH), lambda i, j: (i, 0, j)),
        out_shape=jax.ShapeDtypeStruct((B, MAX_OBJ, PIXP), jnp.float32),
    )(proto_t, fc, fb)
    return masks[:, :, :PIX].reshape(B, MAX_OBJ, H, W), fl, fs


# speedup vs baseline: 14.0016x; 6.5144x over previous
import functools

import jax
import jax.numpy as jnp
from jax import lax
from jax.experimental import pallas as pl
from jax.experimental.pallas import tpu as pltpu
from jax.experimental.pallas import tpu_sc as plsc

B, N, C, K, H, W = 16, 18525, 81, 32, 136, 136
TOPN, MAX_OBJ = 200, 100
MIN_SCORE, NMS_THR = 0.05, 0.5


CH = 1544
NBLK = 12
NP = CH * NBLK


def _stage1a_body(cls_ref, box_ref, anc_ref, p_out, boxes_out):
    j = pl.program_id(1)
    row0 = j * CH
    x = cls_ref[0]
    xm = jnp.max(x, axis=1, keepdims=True)
    e = jnp.exp(x - xm)
    s = jnp.sum(e, axis=1, keepdims=True)
    p = e / s
    li = jax.lax.broadcasted_iota(jnp.int32, (CH, C), 1)
    pm = jnp.where(li >= 1, p, 0.0)
    valid = jnp.max(pm, axis=1, keepdims=True) > MIN_SCORE
    ri = row0 + jax.lax.broadcasted_iota(jnp.int32, (CH, C), 0)
    p_out[0] = jnp.where(ri < N, pm * valid.astype(pm.dtype), 0.0)

    bp = box_ref[0]
    anc = anc_ref[...]
    xy = anc[:, :2] + bp[:, :2] * 0.1 * anc[:, 2:4]
    wh = anc[:, 2:4] * jnp.exp(bp[:, 2:4] * 0.2)
    x1y1 = xy - wh / 2.0
    bx = jnp.clip(jnp.concatenate([x1y1, x1y1 + wh], axis=1), 0.0, 1.0)
    ri4 = row0 + jax.lax.broadcasted_iota(jnp.int32, (CH, 4), 0)
    boxes_out[0] = jnp.where(ri4 < N, bx, 0.0)


def _stage1b_body(p_hbm, meta_out, scratch, sem, *, topn=TOPN):
    b = pl.program_id(0)
    cp = pltpu.make_async_copy(p_hbm.at[b], scratch, sem)
    cp.start()
    cp.wait()

    def count_gt(tf):
        def blk(k, acc):
            ch = scratch[pl.ds(k * CH, CH), :]
            return acc + jnp.sum((ch > tf).astype(jnp.int32), axis=0,
                                 keepdims=True)
        return jax.lax.fori_loop(0, NBLK, blk, jnp.zeros((1, C), jnp.int32))

    one_bits = jnp.int32(0x3F800000)
    lo0 = jnp.zeros((1, C), jnp.int32)
    hi0 = jnp.full((1, C), one_bits, jnp.int32)

    def bs_body(_, lohi):
        lo, hi = lohi
        mid = (lo + hi) >> 1
        midf = jax.lax.bitcast_convert_type(mid, jnp.float32)
        pred = count_gt(midf) >= topn
        return jnp.where(pred, mid, lo), jnp.where(pred, hi, mid)

    lo, hi = jax.lax.fori_loop(0, 31, bs_body, (lo0, hi0))
    cnt0 = count_gt(jnp.zeros((1, C), jnp.float32))
    v200b = jnp.where(cnt0 >= topn, hi, 0)
    v200f = jax.lax.bitcast_convert_type(v200b, jnp.float32)
    m = count_gt(v200f)
    r = topn - m

    zero = jnp.zeros((1, C), jnp.int32)
    meta_out[0] = jnp.concatenate(
        [v200b, r, m, zero, zero, zero, zero, zero], axis=0)


def _stage1_body(cls_ref, box_ref, anc_ref, p_out, meta_out, boxes_out,
                 *, n=N, topn=TOPN):
    x = cls_ref[0]
    xm = jnp.max(x, axis=1, keepdims=True)
    e = jnp.exp(x - xm)
    s = jnp.sum(e, axis=1, keepdims=True)
    p = e / s
    li = jax.lax.broadcasted_iota(jnp.int32, (n, C), 1)
    pm = jnp.where(li >= 1, p, 0.0)
    valid = jnp.max(pm, axis=1, keepdims=True) > MIN_SCORE
    pmm = pm * valid.astype(pm.dtype)
    p_out[0] = pmm

    bp = box_ref[0]
    anc = anc_ref[...]
    xy = anc[:, :2] + bp[:, :2] * 0.1 * anc[:, 2:4]
    wh = anc[:, 2:4] * jnp.exp(bp[:, 2:4] * 0.2)
    x1y1 = xy - wh / 2.0
    boxes_out[0] = jnp.clip(jnp.concatenate([x1y1, x1y1 + wh], axis=1), 0.0, 1.0)

    def count_gt(tf):
        return jnp.sum((pmm > tf).astype(jnp.int32), axis=0, keepdims=True)

    one_bits = jnp.int32(0x3F800000)
    lo0 = jnp.zeros((1, C), jnp.int32)
    hi0 = jnp.full((1, C), one_bits, jnp.int32)

    def bs_body(_, lohi):
        lo, hi = lohi
        mid = (lo + hi) >> 1
        midf = jax.lax.bitcast_convert_type(mid, jnp.float32)
        pred = count_gt(midf) >= topn
        return jnp.where(pred, mid, lo), jnp.where(pred, hi, mid)

    lo, hi = jax.lax.fori_loop(0, 31, bs_body, (lo0, hi0))
    cnt0 = count_gt(jnp.zeros((1, C), jnp.float32))
    v200b = jnp.where(cnt0 >= topn, hi, 0)
    v200f = jax.lax.bitcast_convert_type(v200b, jnp.float32)
    m = count_gt(v200f)
    r = topn - m

    eq = pmm == v200f
    ai = jax.lax.broadcasted_iota(jnp.int32, (n, C), 0)

    def bsI_body(_, lohi):
        lo, hi = lohi
        mid = (lo + hi) >> 1
        cnt = jnp.sum((eq & (ai <= mid)).astype(jnp.int32), axis=0, keepdims=True)
        pred = cnt >= r
        return jnp.where(pred, lo, mid), jnp.where(pred, mid, hi)

    loI0 = jnp.full((1, C), -1, jnp.int32)
    hiI0 = jnp.full((1, C), n - 1, jnp.int32)
    loI, hiI = jax.lax.fori_loop(0, 15, bsI_body, (loI0, hiI0))
    Ilim = jnp.where(r > 0, hiI, -1)

    zero = jnp.zeros((1, C), jnp.int32)
    meta_out[0] = jnp.concatenate(
        [v200b, Ilim, r, m, zero, zero, zero, zero], axis=0)


NCLS = C - 1
NWORK = 32
TASKS = B * NCLS
TPW = TASKS // NWORK
NVR = NP // 16


CPW = NCLS // 2


def _k2_body(cls_t_hbm, thr_hbm, r_hbm, boxes_hbm,
             oi_hbm, ov_hbm, ob_hbm,
             row_v, boxes_v, oi_v, ov_v, ob0_v, ob1_v, ob2_v, ob3_v,
             thr_v, r_v):
    ob_v = (ob0_v, ob1_v, ob2_v, ob3_v)
    wid = lax.axis_index("s") * 2 + lax.axis_index("c")
    b = wid // 2
    c0 = (wid % 2) * CPW
    pltpu.sync_copy(thr_hbm, thr_v)
    pltpu.sync_copy(r_hbm, r_v)
    pltpu.sync_copy(boxes_hbm.at[b], boxes_v)
    lanes = lax.iota(jnp.int32, 16)

    def task_body(t, _):
        c = c0 + t + 1
        pltpu.sync_copy(cls_t_hbm.at[b, c], row_v)
        code = jnp.full((16,), b * C + c, jnp.int32)
        thrv = plsc.load_gather(thr_v, [code])
        rv = plsc.load_gather(r_v, [code])

        def vloop(k2, carry):
            ptr, eqseen = carry
            v = row_v[pl.ds(k2 * 16, 16)]
            idxv = lanes + k2 * 16
            gt = v > thrv
            eq = v == thrv
            eqc = jax.lax.cumsum(eq.astype(jnp.int32))
            take = gt | (eq & ((eqc + (eqseen - 1)) < rv))
            nsel = jnp.sum(take.astype(jnp.int32))
            neq = jnp.sum(eq.astype(jnp.int32))
            plsc.store_compressed(oi_v.at[pl.ds(ptr, 16)], idxv, mask=take)
            plsc.store_compressed(ov_v.at[pl.ds(ptr, 16)], v, mask=take)
            return ptr + nsel, eqseen + neq

        lax.fori_loop(0, NVR, vloop, (jnp.int32(0), jnp.int32(0)))

        def gloop(k2, _):
            idx = oi_v[pl.ds(k2 * 16, 16)]
            base = jnp.minimum(jnp.maximum(idx, 0), NP - 1) * 4
            for comp in range(4):
                g = plsc.load_gather(boxes_v, [base + comp])
                ob_v[comp][pl.ds(k2 * 16, 16)] = g
            return 0

        lax.fori_loop(0, (TOPN + 15) // 16, gloop, 0)
        pltpu.sync_copy(oi_v, oi_hbm.at[b, c - 1])
        pltpu.sync_copy(ov_v, ov_hbm.at[b, c - 1])
        for comp in range(4):
            pltpu.sync_copy(ob_v[comp], ob_hbm.at[b, c - 1, comp])
        return 0

    lax.fori_loop(0, CPW, task_body, 0)


def _k2_call(cls_t, thr_flat, r_flat, boxes_flat):
    return pl.kernel(
        _k2_body,
        out_type=[
            jax.ShapeDtypeStruct((B, NCLS, 256), jnp.int32),
            jax.ShapeDtypeStruct((B, NCLS, 256), jnp.float32),
            jax.ShapeDtypeStruct((B, NCLS, 4, 256), jnp.float32),
        ],
        mesh=plsc.VectorSubcoreMesh(core_axis_name="c", subcore_axis_name="s"),
        scratch_types=[
            pltpu.VMEM((NP,), jnp.float32),
            pltpu.VMEM((NP * 4,), jnp.float32),
            pltpu.VMEM((256,), jnp.int32),
            pltpu.VMEM((256,), jnp.float32),
            pltpu.VMEM((256,), jnp.float32),
            pltpu.VMEM((256,), jnp.float32),
            pltpu.VMEM((256,), jnp.float32),
            pltpu.VMEM((256,), jnp.float32),
            pltpu.VMEM((B * C,), jnp.float32),
            pltpu.VMEM((B * C,), jnp.int32),
        ],
        compiler_params=pltpu.CompilerParams(needs_layout_passes=False),
    )(cls_t, thr_flat, r_flat, boxes_flat)


PIX = H * W
PIXP = 18560
PCH = 3712
NPBLK = 5


def _stage5_body(proto_ref, fc_ref, fb_ref, out_ref):
    j = pl.program_id(1)
    fc = fc_ref[0]
    pt = proto_ref[0]
    logit = jnp.dot(fc, pt)
    pix = j * PCH + jax.lax.broadcasted_iota(jnp.int32, (MAX_OBJ, PCH), 1)
    px = (pix % W).astype(jnp.float32)
    py = (pix // W).astype(jnp.float32)
    fb = fb_ref[0]
    x1 = fb[:, 0:1] * W
    y1 = fb[:, 1:2] * H
    x2 = fb[:, 2:3] * W
    y2 = fb[:, 3:4] * H
    inside = (px >= x1) & (px < x2) & (py >= y1) & (py < y2)
    out_ref[0] = ((logit > 0.0) & inside).astype(jnp.float32)


def _pairwise_iou(b):
    x1 = jnp.maximum(b[:, :, None, 0], b[:, None, :, 0])
    y1 = jnp.maximum(b[:, :, None, 1], b[:, None, :, 1])
    x2 = jnp.minimum(b[:, :, None, 2], b[:, None, :, 2])
    y2 = jnp.minimum(b[:, :, None, 3], b[:, None, :, 3])
    inter = jnp.clip(x2 - x1, 0.0) * jnp.clip(y2 - y1, 0.0)
    area = (b[..., 2] - b[..., 0]) * (b[..., 3] - b[..., 1])
    union = area[:, :, None] + area[:, None, :] - inter
    return inter / jnp.maximum(union, 1e-9)


def _decode_from_sel(sel_idx, v, b, coef_p):
    co = coef_p[jnp.minimum(sel_idx, N - 1)]

    iou = _pairwise_iou(b)
    prec = (v[:, :, None] > v[:, None, :]) | (
        (v[:, :, None] == v[:, None, :]) & (sel_idx[:, :, None] < sel_idx[:, None, :]))
    suppressed = jnp.any(prec & (iou > NMS_THR), axis=1)
    keep = ~suppressed

    scores_f = (v * keep.astype(v.dtype) * (v > MIN_SCORE).astype(v.dtype)).reshape(-1)
    fs, fi = lax.top_k(scores_f, MAX_OBJ)
    fb = b.reshape(-1, 4)[fi]
    fc = co.reshape(-1, K)[fi]
    fl = fi // TOPN
    return fb, fc, fl.astype(jnp.int32), fs


def kernel(class_preds, box_preds, coef_preds, proto_outs, anchors):
    p_pad, boxes_pad = pl.pallas_call(
        _stage1a_body,
        grid=(B, NBLK),
        in_specs=[
            pl.BlockSpec((1, CH, C), lambda i, j: (i, j, 0)),
            pl.BlockSpec((1, CH, 4), lambda i, j: (i, j, 0)),
            pl.BlockSpec((CH, 4), lambda i, j: (j, 0)),
        ],
        out_specs=[
            pl.BlockSpec((1, CH, C), lambda i, j: (i, j, 0)),
            pl.BlockSpec((1, CH, 4), lambda i, j: (i, j, 0)),
        ],
        out_shape=[
            jax.ShapeDtypeStruct((B, NP, C), jnp.float32),
            jax.ShapeDtypeStruct((B, NP, 4), jnp.float32),
        ],
    )(class_preds, box_preds, anchors)

    meta = pl.pallas_call(
        _stage1b_body,
        grid=(B,),
        in_specs=[pl.BlockSpec(memory_space=pltpu.MemorySpace.HBM)],
        out_specs=pl.BlockSpec((1, 8, C), lambda i: (i, 0, 0)),
        out_shape=jax.ShapeDtypeStruct((B, 8, C), jnp.int32),
        scratch_shapes=[
            pltpu.VMEM((NP, C), jnp.float32),
            pltpu.SemaphoreType.DMA,
        ],
    )(p_pad)

    cls_t = p_pad.transpose(0, 2, 1)
    thr_flat = jax.lax.bitcast_convert_type(meta[:, 0, :], jnp.float32).reshape(-1)
    r_flat = meta[:, 1, :].reshape(-1)
    oi, ov, ob = _k2_call(cls_t, thr_flat, r_flat, boxes_pad.reshape(B, NP * 4))
    fb, fc, fl, fs = jax.vmap(_decode_from_sel)(
        oi[..., :TOPN], ov[..., :TOPN],
        ob[..., :TOPN].transpose(0, 1, 3, 2), coef_preds)

    proto_t = proto_outs.reshape(B, PIX, K).transpose(0, 2, 1)
    proto_t = jnp.pad(proto_t, ((0, 0), (0, 0), (0, PIXP - PIX)))
    masks = pl.pallas_call(
        _stage5_body,
        grid=(B, NPBLK),
        in_specs=[
            pl.BlockSpec((1, K, PCH), lambda i, j: (i, 0, j)),
            pl.BlockSpec((1, MAX_OBJ, K), lambda i, j: (i, 0, 0)),
            pl.BlockSpec((1, MAX_OBJ, 4), lambda i, j: (i, 0, 0)),
        ],
        out_specs=pl.BlockSpec((1, MAX_OBJ, PCH), lambda i, j: (i, 0, j)),
        out_shape=jax.ShapeDtypeStruct((B, MAX_OBJ, PIXP), jnp.float32),
    )(proto_t, fc, fb)
    return masks[:, :, :PIX].reshape(B, MAX_OBJ, H, W), fl, fs
